# Initial kernel scaffold; baseline (speedup 1.0000x reference)
import jax
import jax.numpy as jnp
from jax.experimental import pallas as pl


def kernel(input, table):
    raise NotImplementedError("write your pallas kernel here")


# speedup vs baseline: 41.9363x; 41.9363x over previous
import functools

import jax
import jax.numpy as jnp
from jax import lax
from jax.experimental import pallas as pl
from jax.experimental.pallas import tpu as pltpu
from jax.experimental.pallas import tpu_sc as plsc

NUM_BUCKETS = 100000
EMB = 16
BATCH = 16384
MAX_LEN = 100

NC = 2
NS = 16
NW = NC * NS
ROWS_W = BATCH // NW
NBUF = 4
NGRP = ROWS_W // NBUF


def _reduce_row(rows_v, b):
    accs = [rows_v[b, j, :] for j in range(4)]
    for l in range(4, MAX_LEN, 4):
        for j in range(4):
            accs[j] = accs[j] + rows_v[b, l + j, :]
    return (accs[0] + accs[1]) + (accs[2] + accs[3])


def _body(inp_hbm, tbl_hbm, out_hbm, idx_v, rows_v, out_v, s0, s1, s2, s3):
    sems = (s0, s1, s2, s3)
    wid = lax.axis_index("s") * NC + lax.axis_index("c")
    base = wid * ROWS_W

    pltpu.sync_copy(inp_hbm.at[pl.ds(base, ROWS_W)], idx_v)

    for b in range(NBUF):
        pltpu.async_copy(tbl_hbm.at[idx_v.at[b]], rows_v.at[b], sems[b])

    def grp(g, c):
        for b in range(NBUF):
            r = g * NBUF + b
            pltpu.make_async_copy(
                tbl_hbm.at[idx_v.at[r]], rows_v.at[b], sems[b]).wait()
            acc = _reduce_row(rows_v, b)
            out_v[r, :] = acc * jnp.float32(1.0 / MAX_LEN)
            pltpu.async_copy(
                tbl_hbm.at[idx_v.at[r + NBUF]], rows_v.at[b], sems[b])
        return c

    lax.fori_loop(0, NGRP - 1, grp, 0)

    for b in range(NBUF):
        r = (NGRP - 1) * NBUF + b
        pltpu.make_async_copy(
            tbl_hbm.at[idx_v.at[r]], rows_v.at[b], sems[b]).wait()
        acc = _reduce_row(rows_v, b)
        out_v[r, :] = acc * jnp.float32(1.0 / MAX_LEN)

    pltpu.sync_copy(out_v, out_hbm.at[pl.ds(base, ROWS_W)])


_embed = functools.partial(
    pl.kernel,
    out_type=jax.ShapeDtypeStruct((BATCH, EMB), jnp.float32),
    mesh=plsc.VectorSubcoreMesh(core_axis_name="c", subcore_axis_name="s"),
    compiler_params=pltpu.CompilerParams(use_tc_tiling_on_sc=False),
    scratch_types=[
        pltpu.VMEM((ROWS_W, MAX_LEN), jnp.int32),
        pltpu.VMEM((NBUF, MAX_LEN, EMB), jnp.float32),
        pltpu.VMEM((ROWS_W, EMB), jnp.float32),
        pltpu.SemaphoreType.DMA,
        pltpu.SemaphoreType.DMA,
        pltpu.SemaphoreType.DMA,
        pltpu.SemaphoreType.DMA,
    ],
)(_body)


def kernel(input, table):
    return _embed(input, table)


# speedup vs baseline: 46.2464x; 1.1028x over previous
import functools

import jax
import jax.numpy as jnp
from jax import lax
from jax.experimental import pallas as pl
from jax.experimental.pallas import tpu as pltpu
from jax.experimental.pallas import tpu_sc as plsc

NUM_BUCKETS = 100000
EMB = 16
BATCH = 16384
MAX_LEN = 100

NC = 2
NS = 16
NW = NC * NS
ROWS_W = BATCH // NW
PAIRS = ROWS_W // 2
NIDX = 8
NGAT = 4
UNROLL = 8
ROWS_STAGE = NUM_BUCKETS // NS


def _reduce_row(rows_v, q, j):
    accs = [rows_v[q, j, k, :] for k in range(4)]
    for l in range(4, MAX_LEN, 4):
        for k in range(4):
            accs[k] = accs[k] + rows_v[q, j, l + k, :]
    return (accs[0] + accs[1]) + (accs[2] + accs[3])


def _body(inp_hbm, tbl_hbm, out_hbm, tbl_s, idx_v, rows_v, out_v, *sems):
    isems = sems[:NIDX]
    gsems = sems[NIDX:NIDX + NGAT]
    cid = lax.axis_index("c")
    sid = lax.axis_index("s")
    wid = sid * NC + cid
    base = wid * ROWS_W
    pbase = base // 2

    pltpu.sync_copy(tbl_hbm.at[pl.ds(sid * ROWS_STAGE, ROWS_STAGE)],
                    tbl_s.at[pl.ds(sid * ROWS_STAGE, ROWS_STAGE)])

    def fetch_idx(p, slot):
        pltpu.async_copy(inp_hbm.at[pl.ds((pbase + p) * 2, 2)],
                         idx_v.at[slot], isems[slot])

    def fire_pair(p, islot, gslot):
        pltpu.make_async_copy(inp_hbm.at[pl.ds(0, 2)], idx_v.at[islot],
                              isems[islot]).wait()
        for j in range(2):
            pltpu.async_copy(tbl_s.at[idx_v.at[islot, j]],
                             rows_v.at[gslot, j], gsems[gslot])

    def drain_pair(p, gslot):
        for j in range(2):
            pltpu.make_async_copy(tbl_hbm.at[pl.ds(0, MAX_LEN)],
                                  rows_v.at[gslot, j], gsems[gslot]).wait()
        for j in range(2):
            acc = _reduce_row(rows_v, gslot, j)
            out_v[p * 2 + j, :] = acc * jnp.float32(1.0 / MAX_LEN)

    for p in range(NIDX):
        fetch_idx(p, p)
    plsc.subcore_barrier()
    for p in range(NGAT):
        fire_pair(p, p % NIDX, p % NGAT)

    def grp(g, c):
        for j in range(UNROLL):
            p = g * UNROLL + j
            drain_pair(p, j % NGAT)

            @pl.when(p + NIDX < PAIRS)
            def _():
                fetch_idx(p + NIDX, j % NIDX)

            @pl.when(p + NGAT < PAIRS)
            def _():
                fire_pair(p + NGAT, (j + NGAT) % NIDX, j % NGAT)
        return c

    lax.fori_loop(0, PAIRS // UNROLL, grp, 0)

    pltpu.sync_copy(out_v, out_hbm.at[pl.ds(base, ROWS_W)])


_embed = functools.partial(
    pl.kernel,
    out_type=jax.ShapeDtypeStruct((BATCH, EMB), jnp.float32),
    mesh=plsc.VectorSubcoreMesh(core_axis_name="c", subcore_axis_name="s"),
    compiler_params=pltpu.CompilerParams(use_tc_tiling_on_sc=False),
    scratch_types=[
        pltpu.VMEM_SHARED((NUM_BUCKETS, EMB), jnp.float32),
        pltpu.VMEM((NIDX, 2, MAX_LEN), jnp.int32),
        pltpu.VMEM((NGAT, 2, MAX_LEN, EMB), jnp.float32),
        pltpu.VMEM((ROWS_W, EMB), jnp.float32),
    ] + [pltpu.SemaphoreType.DMA] * (NIDX + NGAT),
)(_body)


def kernel(input, table):
    return _embed(input, table)


# speedup vs baseline: 48.8094x; 1.0554x over previous
import functools

import jax
import jax.numpy as jnp
from jax import lax
from jax.experimental import pallas as pl
from jax.experimental.pallas import tpu as pltpu
from jax.experimental.pallas import tpu_sc as plsc

NUM_BUCKETS = 100000
EMB = 16
BATCH = 16384
MAX_LEN = 100
LANE = 128

NC = 2
NS = 16
NW = NC * NS
ROWS_W = BATCH // NW
PAIRS = ROWS_W // 2
NIDX = 8
NGAT = 4
UNROLL = 8
ROWS_STAGE = NUM_BUCKETS // NS
GLEN = 104


def _reduce_row(rows_v, q, j):
    accs = [rows_v[q, j, k, :] for k in range(4)]
    for l in range(4, MAX_LEN, 4):
        for k in range(4):
            accs[k] = accs[k] + rows_v[q, j, l + k, :]
    return (accs[0] + accs[1]) + (accs[2] + accs[3])


def _body(inp_hbm, tbl_hbm, out_hbm, tbl_s, idx_v, rows_v, out_v, *sems):
    isems = sems[:NIDX]
    gsems = sems[NIDX:NIDX + NGAT]
    cid = lax.axis_index("c")
    sid = lax.axis_index("s")
    wid = sid * NC + cid
    base = wid * ROWS_W
    pbase = base // 2

    pltpu.sync_copy(tbl_hbm.at[pl.ds(sid * ROWS_STAGE, ROWS_STAGE)],
                    tbl_s.at[pl.ds(sid * ROWS_STAGE, ROWS_STAGE)])

    def fetch_idx(p, slot):
        pltpu.async_copy(inp_hbm.at[pl.ds((pbase + p) * 2, 2)],
                         idx_v.at[slot], isems[slot])

    def fire_pair(p, islot, gslot):
        pltpu.make_async_copy(inp_hbm.at[pl.ds(0, 2)], idx_v.at[islot],
                              isems[islot]).wait()
        for j in range(2):
            pltpu.async_copy(tbl_s.at[idx_v.at[islot, j, pl.ds(0, GLEN)]],
                             rows_v.at[gslot, j], gsems[gslot])

    def drain_pair(p, gslot):
        for j in range(2):
            pltpu.make_async_copy(tbl_s.at[pl.ds(0, GLEN)],
                                  rows_v.at[gslot, j], gsems[gslot]).wait()
        for j in range(2):
            acc = _reduce_row(rows_v, gslot, j)
            r = p * 2 + j
            out_v[r // 8, pl.ds((r % 8) * EMB, EMB)] = (
                acc * jnp.float32(1.0 / MAX_LEN))

    for p in range(NIDX):
        fetch_idx(p, p)
    plsc.subcore_barrier()
    for p in range(NGAT):
        fire_pair(p, p % NIDX, p % NGAT)

    def grp(g, c):
        for j in range(UNROLL):
            p = g * UNROLL + j
            drain_pair(p, j % NGAT)

            @pl.when(p + NIDX < PAIRS)
            def _():
                fetch_idx(p + NIDX, j % NIDX)

            @pl.when(p + NGAT < PAIRS)
            def _():
                fire_pair(p + NGAT, (j + NGAT) % NIDX, j % NGAT)
        return c

    lax.fori_loop(0, PAIRS // UNROLL, grp, 0)

    pltpu.sync_copy(out_v, out_hbm.at[pl.ds(wid * (ROWS_W * EMB // LANE),
                                            ROWS_W * EMB // LANE)])


_embed = functools.partial(
    pl.kernel,
    out_type=jax.ShapeDtypeStruct((BATCH * EMB // LANE, LANE), jnp.float32),
    mesh=plsc.VectorSubcoreMesh(core_axis_name="c", subcore_axis_name="s"),
    compiler_params=pltpu.CompilerParams(use_tc_tiling_on_sc=False),
    scratch_types=[
        pltpu.VMEM_SHARED((NUM_BUCKETS, EMB), jnp.float32),
        pltpu.VMEM((NIDX, 2, LANE), jnp.int32),
        pltpu.VMEM((NGAT, 2, GLEN, EMB), jnp.float32),
        pltpu.VMEM((ROWS_W * EMB // LANE, LANE), jnp.float32),
    ] + [pltpu.SemaphoreType.DMA] * (NIDX + NGAT),
)(_body)


def kernel(input, table):
    inp_p = jnp.pad(input, ((0, 0), (0, LANE - MAX_LEN)))
    out = _embed(inp_p, table)
    return out.reshape(BATCH, EMB)
